# Initial kernel scaffold; baseline (speedup 1.0000x reference)
import jax
import jax.numpy as jnp
from jax.experimental import pallas as pl


def kernel(x, edge_index, edge_weight, W0, b0, W1, b1, W2, b2):
    raise NotImplementedError("write your pallas kernel here")


# speedup vs baseline: 22.7961x; 22.7961x over previous
import functools

import jax
import jax.numpy as jnp
from jax import lax
from jax.experimental import pallas as pl
from jax.experimental.pallas import tpu as pltpu
from jax.experimental.pallas import tpu_sc as plsc

N = 10000
E = 320000
D = 128
NC = 2
NS = 16
NW = NC * NS
EPT = E // NW
CHUNK = 80
NCHUNK = EPT // CHUNK
ROWS_PT = N // NS

f32 = jnp.float32
i32 = jnp.int32


def _k1_body(ew_ref, mn_ref, inv_ref):
    ew = ew_ref[...]
    mn = jnp.min(ew)
    mx = jnp.max(ew)
    mn_ref[0, 0] = mn
    inv_ref[0, 0] = 1.0 / (mx - mn)


def _k1(ew2d):
    return pl.pallas_call(
        _k1_body,
        out_shape=[jax.ShapeDtypeStruct((1, 1), f32),
                   jax.ShapeDtypeStruct((1, 1), f32)],
        out_specs=[pl.BlockSpec(memory_space=pltpu.SMEM),
                   pl.BlockSpec(memory_space=pltpu.SMEM)],
    )(ew2d)


def _k2_body(dst_hbm, ew_hbm, mn_hbm, inv_hbm, degp_hbm,
             deg_v, dst_v, ew_v, mn_v, inv_v):
    c = lax.axis_index("c")
    s = lax.axis_index("s")
    wid = c * NS + s
    base = wid * EPT

    @pl.loop(0, N // 16, unroll=4)
    def _zero(k):
        deg_v[pl.ds(k * 16, 16)] = jnp.zeros((16,), f32)

    pltpu.sync_copy(mn_hbm, mn_v)
    pltpu.sync_copy(inv_hbm, inv_v)
    mn = mn_v[...]
    inv = inv_v[...]

    CB = 2000
    for chunk in range(EPT // CB):
        off = base + chunk * CB
        pltpu.sync_copy(dst_hbm.at[pl.ds(off, CB)], dst_v)
        pltpu.sync_copy(ew_hbm.at[pl.ds(off, CB)], ew_v)

        @pl.loop(0, CB // 16, unroll=4)
        def _acc(g):
            dd = dst_v[pl.ds(g * 16, 16)]
            w = (ew_v[pl.ds(g * 16, 16)] - mn) * inv
            plsc.addupdate_scatter(deg_v, [dd], w)

    pltpu.sync_copy(deg_v, degp_hbm.at[wid])


def _k2(dst, ew, mn16, inv16):
    mesh = plsc.VectorSubcoreMesh(core_axis_name="c", subcore_axis_name="s",
                                  num_cores=NC, num_subcores=NS)
    kfn = pl.kernel(
        _k2_body,
        out_type=jax.ShapeDtypeStruct((NW, N), f32),
        mesh=mesh,
        compiler_params=pltpu.CompilerParams(needs_layout_passes=False, use_tc_tiling_on_sc=False),
        scratch_types=[
            pltpu.VMEM((N,), f32),
            pltpu.VMEM((2000,), i32),
            pltpu.VMEM((2000,), f32),
            pltpu.VMEM((16,), f32),
            pltpu.VMEM((16,), f32),
        ],
    )
    return kfn(dst, ew, mn16, inv16)


def _k3_body(degp_ref, dinv_ref):
    deg = jnp.sum(degp_ref[...], axis=0, keepdims=True) + 1.0
    dinv_ref[...] = lax.rsqrt(deg)


def _k3(degp):
    return pl.pallas_call(
        _k3_body,
        out_shape=jax.ShapeDtypeStruct((1, N), f32),
    )(degp)


def _k4_body(x_hbm, src_hbm, dst_hbm, ew_hbm, mn_hbm, inv_hbm, dinv_hbm,
             ypart_hbm,
             dinv_v, src_v, dst_v, ew_v, norm_v, rows_v, zbuf_v,
             mn_v, inv_v, y_sh, sem):
    c = lax.axis_index("c")
    s = lax.axis_index("s")
    wid = c * NS + s
    base = wid * EPT

    @pl.loop(0, 125, unroll=2)
    def _zzero(k):
        for j in range(D // 16):
            zbuf_v[k, pl.ds(j * 16, 16)] = jnp.zeros((16,), f32)

    for zi in range(ROWS_PT // 125):
        pltpu.sync_copy(zbuf_v, y_sh.at[pl.ds(s * ROWS_PT + zi * 125, 125)])

    pltpu.sync_copy(mn_hbm, mn_v)
    pltpu.sync_copy(inv_hbm, inv_v)
    pltpu.sync_copy(dinv_hbm, dinv_v)
    mn = mn_v[...]
    inv = inv_v[...]

    plsc.subcore_barrier()

    @pl.loop(0, NCHUNK)
    def _chunk(ci):
        off = base + ci * CHUNK
        pltpu.sync_copy(src_hbm.at[pl.ds(off, CHUNK)], src_v)
        pltpu.sync_copy(dst_hbm.at[pl.ds(off, CHUNK)], dst_v)
        pltpu.sync_copy(ew_hbm.at[pl.ds(off, CHUNK)], ew_v)
        pltpu.async_copy(x_hbm.at[src_v], rows_v, sem).wait()

        for g in range(CHUNK // 16):
            sv = src_v[pl.ds(g * 16, 16)]
            dv = dst_v[pl.ds(g * 16, 16)]
            w = (ew_v[pl.ds(g * 16, 16)] - mn) * inv
            dis = plsc.load_gather(dinv_v, [sv])
            did = plsc.load_gather(dinv_v, [dv])
            norm_v[pl.ds(g * 16, 16)] = dis * w * did

        @pl.loop(0, CHUNK, unroll=2)
        def _scale(r):
            nb = plsc.load_gather(norm_v, [jnp.full((16,), r, i32)])
            for j in range(D // 16):
                rows_v[r, pl.ds(j * 16, 16)] = rows_v[r, pl.ds(j * 16, 16)] * nb

        pltpu.sync_copy(rows_v, y_sh.at[dst_v], add=True)

    plsc.subcore_barrier()

    pltpu.sync_copy(y_sh.at[pl.ds(s * ROWS_PT, ROWS_PT)],
                    ypart_hbm.at[c, pl.ds(s * ROWS_PT, ROWS_PT)])


def _k4(x, src, dst, ew, mn16, inv16, dinv):
    mesh = plsc.VectorSubcoreMesh(core_axis_name="c", subcore_axis_name="s",
                                  num_cores=NC, num_subcores=NS)
    kfn = pl.kernel(
        _k4_body,
        out_type=jax.ShapeDtypeStruct((NC, N, D), f32),
        mesh=mesh,
        compiler_params=pltpu.CompilerParams(needs_layout_passes=False, use_tc_tiling_on_sc=False),
        scratch_types=[
            pltpu.VMEM((N,), f32),
            pltpu.VMEM((CHUNK,), i32),
            pltpu.VMEM((CHUNK,), i32),
            pltpu.VMEM((CHUNK,), f32),
            pltpu.VMEM((CHUNK,), f32),
            pltpu.VMEM((CHUNK, D), f32),
            pltpu.VMEM((125, D), f32),
            pltpu.VMEM((16,), f32),
            pltpu.VMEM((16,), f32),
            pltpu.VMEM_SHARED((N, D), f32),
            pltpu.SemaphoreType.DMA,
        ],
    )
    return kfn(x, src, dst, ew, mn16, inv16, dinv)


def _k5_body(yp_ref, x_ref, dinv_ref, w0_ref, b0_ref, w1_ref, b1_ref,
             w2_ref, b2_ref, out_ref):
    d = dinv_ref[...]
    y = yp_ref[0] + yp_ref[1] + x_ref[...] * (d * d)
    out_ref[:, 0:D] = jnp.dot(y, w0_ref[...], preferred_element_type=f32) + b0_ref[...]
    out_ref[:, D:2 * D] = jnp.dot(y, w1_ref[...], preferred_element_type=f32) + b1_ref[...]
    out_ref[:, 2 * D:3 * D] = jnp.dot(y, w2_ref[...], preferred_element_type=f32) + b2_ref[...]


def _k5(ypart, x, dinv2d, W0, b0, W1, b1, W2, b2):
    R = 1000
    grid = N // R
    wspec = pl.BlockSpec((D, D), lambda i: (0, 0))
    bspec = pl.BlockSpec((1, D), lambda i: (0, 0))
    return pl.pallas_call(
        _k5_body,
        grid=(grid,),
        in_specs=[
            pl.BlockSpec((NC, R, D), lambda i: (0, i, 0)),
            pl.BlockSpec((R, D), lambda i: (i, 0)),
            pl.BlockSpec((R, 1), lambda i: (i, 0)),
            wspec, bspec, wspec, bspec, wspec, bspec,
        ],
        out_specs=pl.BlockSpec((R, 3 * D), lambda i: (i, 0)),
        out_shape=jax.ShapeDtypeStruct((N, 3 * D), f32),
    )(ypart, x, dinv2d, W0, b0, W1, b1, W2, b2)


def kernel(x, edge_index, edge_weight, W0, b0, W1, b1, W2, b2):
    src = edge_index[0]
    dst = edge_index[1]

    mn, inv = _k1(edge_weight.reshape(E // D, D))
    mn16 = jnp.broadcast_to(mn.reshape(()), (16,))
    inv16 = jnp.broadcast_to(inv.reshape(()), (16,))

    degp = _k2(dst, edge_weight, mn16, inv16)
    dinv2d = _k3(degp)
    dinv = dinv2d.reshape(N)

    ypart = _k4(x, src, dst, edge_weight, mn16, inv16, dinv)

    return _k5(ypart, x, dinv2d.reshape(N, 1), W0,
               b0.reshape(1, D), W1, b1.reshape(1, D), W2, b2.reshape(1, D))


# speedup vs baseline: 31.8812x; 1.3985x over previous
import functools

import jax
import jax.numpy as jnp
from jax import lax
from jax.experimental import pallas as pl
from jax.experimental.pallas import tpu as pltpu
from jax.experimental.pallas import tpu_sc as plsc

N = 10000
E = 320000
D = 128
NC = 2
NS = 16
NW = NC * NS
EPT = E // NW
CHUNK = 80
NCHUNK = EPT // CHUNK
ROWS_PT = N // NS

f32 = jnp.float32
i32 = jnp.int32


def _k1_body(ew_ref, mn_ref, inv_ref):
    ew = ew_ref[...]
    mn = jnp.min(ew)
    mx = jnp.max(ew)
    mn_ref[0, 0] = mn
    inv_ref[0, 0] = 1.0 / (mx - mn)


def _k1(ew2d):
    return pl.pallas_call(
        _k1_body,
        out_shape=[jax.ShapeDtypeStruct((1, 1), f32),
                   jax.ShapeDtypeStruct((1, 1), f32)],
        out_specs=[pl.BlockSpec(memory_space=pltpu.SMEM),
                   pl.BlockSpec(memory_space=pltpu.SMEM)],
    )(ew2d)


def _k2_body(dst_hbm, ew_hbm, mn_hbm, inv_hbm, degp_hbm,
             deg_v, dst_v, ew_v, mn_v, inv_v):
    c = lax.axis_index("c")
    s = lax.axis_index("s")
    wid = c * NS + s
    base = wid * EPT

    @pl.loop(0, N // 16, unroll=4)
    def _zero(k):
        deg_v[pl.ds(k * 16, 16)] = jnp.zeros((16,), f32)

    pltpu.sync_copy(mn_hbm, mn_v)
    pltpu.sync_copy(inv_hbm, inv_v)
    mn = mn_v[...]
    inv = inv_v[...]

    CB = 2000
    for chunk in range(EPT // CB):
        off = base + chunk * CB
        pltpu.sync_copy(dst_hbm.at[pl.ds(off, CB)], dst_v)
        pltpu.sync_copy(ew_hbm.at[pl.ds(off, CB)], ew_v)

        @pl.loop(0, CB // 16, unroll=4)
        def _acc(g):
            dd = dst_v[pl.ds(g * 16, 16)]
            w = (ew_v[pl.ds(g * 16, 16)] - mn) * inv
            plsc.addupdate_scatter(deg_v, [dd], w)

    pltpu.sync_copy(deg_v, degp_hbm.at[wid])


def _k2(dst, ew, mn16, inv16):
    mesh = plsc.VectorSubcoreMesh(core_axis_name="c", subcore_axis_name="s",
                                  num_cores=NC, num_subcores=NS)
    kfn = pl.kernel(
        _k2_body,
        out_type=jax.ShapeDtypeStruct((NW, N), f32),
        mesh=mesh,
        compiler_params=pltpu.CompilerParams(needs_layout_passes=False, use_tc_tiling_on_sc=False),
        scratch_types=[
            pltpu.VMEM((N,), f32),
            pltpu.VMEM((2000,), i32),
            pltpu.VMEM((2000,), f32),
            pltpu.VMEM((16,), f32),
            pltpu.VMEM((16,), f32),
        ],
    )
    return kfn(dst, ew, mn16, inv16)


def _k3_body(degp_ref, dinv_ref):
    deg = jnp.sum(degp_ref[...], axis=0, keepdims=True) + 1.0
    dinv_ref[...] = lax.rsqrt(deg)


def _k3(degp):
    return pl.pallas_call(
        _k3_body,
        out_shape=jax.ShapeDtypeStruct((1, N), f32),
    )(degp)


def _k4_body(x_hbm, src_hbm, dst_hbm, ew_hbm, mn_hbm, inv_hbm, dinv_hbm,
             ypart_hbm,
             dinv_v, src_v, dst_v, ew_v, rows_v,
             mn_v, inv_v, y_sh, sem):
    c = lax.axis_index("c")
    s = lax.axis_index("s")
    wid = c * NS + s

    @pl.loop(0, CHUNK, unroll=2)
    def _zzero(k):
        for j in range(D // 16):
            rows_v[k, pl.ds(j * 16, 16)] = jnp.zeros((16,), f32)

    for zi in range(7):
        pltpu.sync_copy(rows_v,
                        y_sh.at[pl.ds(s * ROWS_PT + zi * CHUNK, CHUNK)])
    pltpu.sync_copy(rows_v.at[pl.ds(0, ROWS_PT - 7 * CHUNK)],
                    y_sh.at[pl.ds(s * ROWS_PT + 7 * CHUNK,
                                  ROWS_PT - 7 * CHUNK)])

    pltpu.sync_copy(mn_hbm, mn_v)
    pltpu.sync_copy(inv_hbm, inv_v)
    pltpu.sync_copy(dinv_hbm, dinv_v)
    pltpu.sync_copy(src_hbm.at[wid], src_v)
    pltpu.sync_copy(dst_hbm.at[wid], dst_v)
    pltpu.sync_copy(ew_hbm.at[wid], ew_v)
    mn = mn_v[...]
    inv = inv_v[...]

    @pl.loop(0, NCHUNK, unroll=2)
    def _norms(ci):
        for g in range(CHUNK // 16):
            sv = src_v[ci, pl.ds(g * 16, 16)]
            dv = dst_v[ci, pl.ds(g * 16, 16)]
            w = (ew_v[ci, pl.ds(g * 16, 16)] - mn) * inv
            dis = plsc.load_gather(dinv_v, [sv])
            did = plsc.load_gather(dinv_v, [dv])
            ew_v[ci, pl.ds(g * 16, 16)] = dis * w * did

    plsc.subcore_barrier()

    @pl.loop(0, NCHUNK)
    def _chunk(ci):
        pltpu.async_copy(x_hbm.at[src_v.at[ci]], rows_v, sem).wait()

        @pl.loop(0, CHUNK, unroll=2)
        def _scale(r):
            nb = plsc.load_gather(ew_v, [jnp.full((16,), ci, i32),
                                         jnp.full((16,), r, i32)])
            for j in range(D // 16):
                rows_v[r, pl.ds(j * 16, 16)] = rows_v[r, pl.ds(j * 16, 16)] * nb

        pltpu.sync_copy(rows_v, y_sh.at[dst_v.at[ci]], add=True)

    plsc.subcore_barrier()

    pltpu.sync_copy(y_sh.at[pl.ds(s * ROWS_PT, ROWS_PT)],
                    ypart_hbm.at[c, pl.ds(s * ROWS_PT, ROWS_PT)])


def _k4(x, src, dst, ew, mn16, inv16, dinv):
    mesh = plsc.VectorSubcoreMesh(core_axis_name="c", subcore_axis_name="s",
                                  num_cores=NC, num_subcores=NS)
    kfn = pl.kernel(
        _k4_body,
        out_type=jax.ShapeDtypeStruct((NC, N, D), f32),
        mesh=mesh,
        compiler_params=pltpu.CompilerParams(needs_layout_passes=False, use_tc_tiling_on_sc=False),
        scratch_types=[
            pltpu.VMEM((N,), f32),
            pltpu.VMEM((NCHUNK, CHUNK), i32),
            pltpu.VMEM((NCHUNK, CHUNK), i32),
            pltpu.VMEM((NCHUNK, CHUNK), f32),
            pltpu.VMEM((CHUNK, D), f32),
            pltpu.VMEM((16,), f32),
            pltpu.VMEM((16,), f32),
            pltpu.VMEM_SHARED((N, D), f32),
            pltpu.SemaphoreType.DMA,
        ],
    )
    return kfn(x.reshape(N, D),
               src.reshape(NW, NCHUNK, CHUNK),
               dst.reshape(NW, NCHUNK, CHUNK),
               ew.reshape(NW, NCHUNK, CHUNK),
               mn16, inv16, dinv)


def _k5_body(yp_ref, x_ref, dinv_ref, w0_ref, b0_ref, w1_ref, b1_ref,
             w2_ref, b2_ref, out_ref):
    d = dinv_ref[...]
    y = yp_ref[0] + yp_ref[1] + x_ref[...] * (d * d)
    out_ref[:, 0:D] = jnp.dot(y, w0_ref[...], preferred_element_type=f32) + b0_ref[...]
    out_ref[:, D:2 * D] = jnp.dot(y, w1_ref[...], preferred_element_type=f32) + b1_ref[...]
    out_ref[:, 2 * D:3 * D] = jnp.dot(y, w2_ref[...], preferred_element_type=f32) + b2_ref[...]


def _k5(ypart, x, dinv2d, W0, b0, W1, b1, W2, b2):
    R = 1000
    grid = N // R
    wspec = pl.BlockSpec((D, D), lambda i: (0, 0))
    bspec = pl.BlockSpec((1, D), lambda i: (0, 0))
    return pl.pallas_call(
        _k5_body,
        grid=(grid,),
        in_specs=[
            pl.BlockSpec((NC, R, D), lambda i: (0, i, 0)),
            pl.BlockSpec((R, D), lambda i: (i, 0)),
            pl.BlockSpec((R, 1), lambda i: (i, 0)),
            wspec, bspec, wspec, bspec, wspec, bspec,
        ],
        out_specs=pl.BlockSpec((R, 3 * D), lambda i: (i, 0)),
        out_shape=jax.ShapeDtypeStruct((N, 3 * D), f32),
    )(ypart, x, dinv2d, W0, b0, W1, b1, W2, b2)


def kernel(x, edge_index, edge_weight, W0, b0, W1, b1, W2, b2):
    src = edge_index[0]
    dst = edge_index[1]

    mn, inv = _k1(edge_weight.reshape(E // D, D))
    mn16 = jnp.broadcast_to(mn.reshape(()), (16,))
    inv16 = jnp.broadcast_to(inv.reshape(()), (16,))

    degp = _k2(dst, edge_weight, mn16, inv16)
    dinv2d = _k3(degp)
    dinv = dinv2d.reshape(N)

    ypart = _k4(x, src, dst, edge_weight, mn16, inv16, dinv)

    return _k5(ypart, x, dinv2d.reshape(N, 1), W0,
               b0.reshape(1, D), W1, b1.reshape(1, D), W2, b2.reshape(1, D))


# speedup vs baseline: 45.8708x; 1.4388x over previous
import functools

import jax
import jax.numpy as jnp
from jax import lax
from jax.experimental import pallas as pl
from jax.experimental.pallas import tpu as pltpu
from jax.experimental.pallas import tpu_sc as plsc

N = 10000
E = 320000
D = 128
NC = 2
NS = 16
NW = NC * NS
EPT = E // NW
CHUNK = 80
NCHUNK = EPT // CHUNK
ROWS_PT = N // NS

f32 = jnp.float32
i32 = jnp.int32


def _k1_body(ew_ref, mn_ref, inv_ref):
    ew = ew_ref[...]
    mn = jnp.min(ew)
    mx = jnp.max(ew)
    mn_ref[0, 0] = mn
    inv_ref[0, 0] = 1.0 / (mx - mn)


def _k1(ew2d):
    return pl.pallas_call(
        _k1_body,
        out_shape=[jax.ShapeDtypeStruct((1, 1), f32),
                   jax.ShapeDtypeStruct((1, 1), f32)],
        out_specs=[pl.BlockSpec(memory_space=pltpu.SMEM),
                   pl.BlockSpec(memory_space=pltpu.SMEM)],
    )(ew2d)


def _k2_body(dst_hbm, ew_hbm, mn_hbm, inv_hbm, degp_hbm,
             deg_v, dst_v, ew_v, mn_v, inv_v):
    c = lax.axis_index("c")
    s = lax.axis_index("s")
    wid = c * NS + s
    base = wid * EPT

    @pl.loop(0, N // 16, unroll=4)
    def _zero(k):
        deg_v[pl.ds(k * 16, 16)] = jnp.zeros((16,), f32)

    pltpu.sync_copy(mn_hbm, mn_v)
    pltpu.sync_copy(inv_hbm, inv_v)
    mn = mn_v[...]
    inv = inv_v[...]

    CB = 2000
    for chunk in range(EPT // CB):
        off = base + chunk * CB
        pltpu.sync_copy(dst_hbm.at[pl.ds(off, CB)], dst_v)
        pltpu.sync_copy(ew_hbm.at[pl.ds(off, CB)], ew_v)

        @pl.loop(0, CB // 16, unroll=4)
        def _acc(g):
            dd = dst_v[pl.ds(g * 16, 16)]
            w = (ew_v[pl.ds(g * 16, 16)] - mn) * inv
            plsc.addupdate_scatter(deg_v, [dd], w)

    pltpu.sync_copy(deg_v, degp_hbm.at[wid])


def _k2(dst, ew, mn16, inv16):
    mesh = plsc.VectorSubcoreMesh(core_axis_name="c", subcore_axis_name="s",
                                  num_cores=NC, num_subcores=NS)
    kfn = pl.kernel(
        _k2_body,
        out_type=jax.ShapeDtypeStruct((NW, N), f32),
        mesh=mesh,
        compiler_params=pltpu.CompilerParams(needs_layout_passes=False, use_tc_tiling_on_sc=False),
        scratch_types=[
            pltpu.VMEM((N,), f32),
            pltpu.VMEM((2000,), i32),
            pltpu.VMEM((2000,), f32),
            pltpu.VMEM((16,), f32),
            pltpu.VMEM((16,), f32),
        ],
    )
    return kfn(dst, ew, mn16, inv16)


def _k3_body(degp_ref, dinv_ref):
    deg = jnp.sum(degp_ref[...], axis=0, keepdims=True) + 1.0
    dinv_ref[...] = lax.rsqrt(deg)


def _k3(degp):
    return pl.pallas_call(
        _k3_body,
        out_shape=jax.ShapeDtypeStruct((1, N), f32),
    )(degp)


def _k4_body(x_hbm, pk_hbm, ew_hbm, mn_hbm, inv_hbm, dinv_hbm,
             ypart_hbm,
             dinv_v, pk_v, ew_v, rows0_v, rows1_v,
             sb0_v, db0_v, sb1_v, db1_v, mn_v, inv_v, y_sh,
             gsem0, gsem1, ssem0, ssem1):
    c = lax.axis_index("c")
    s = lax.axis_index("s")
    wid = c * NS + s

    def unpack(ci, sb, db):
        for g in range(CHUNK // 16):
            pv = pk_v[ci, pl.ds(g * 16, 16)]
            sb[pl.ds(g * 16, 16)] = jnp.bitwise_and(pv, 0xFFFF)
            db[pl.ds(g * 16, 16)] = jnp.right_shift(pv, 16)

    def scale(rows, ci):
        @pl.loop(0, CHUNK, unroll=2)
        def _scale(r):
            nb = plsc.load_gather(ew_v, [jnp.full((16,), ci, i32),
                                         jnp.full((16,), r, i32)])
            for j in range(D // 16):
                rows[r, pl.ds(j * 16, 16)] = rows[r, pl.ds(j * 16, 16)] * nb

    @pl.loop(0, CHUNK, unroll=2)
    def _zzero(k):
        for j in range(D // 16):
            rows0_v[k, pl.ds(j * 16, 16)] = jnp.zeros((16,), f32)

    for zi in range(7):
        pltpu.sync_copy(rows0_v,
                        y_sh.at[pl.ds(s * ROWS_PT + zi * CHUNK, CHUNK)])
    pltpu.sync_copy(rows0_v.at[pl.ds(0, ROWS_PT - 7 * CHUNK)],
                    y_sh.at[pl.ds(s * ROWS_PT + 7 * CHUNK,
                                  ROWS_PT - 7 * CHUNK)])

    pltpu.sync_copy(mn_hbm, mn_v)
    pltpu.sync_copy(inv_hbm, inv_v)
    pltpu.sync_copy(dinv_hbm, dinv_v)
    pltpu.sync_copy(pk_hbm.at[wid], pk_v)
    pltpu.sync_copy(ew_hbm.at[wid], ew_v)
    mn = mn_v[...]
    inv = inv_v[...]

    @pl.loop(0, NCHUNK, unroll=2)
    def _norms(ci):
        for g in range(CHUNK // 16):
            pv = pk_v[ci, pl.ds(g * 16, 16)]
            sv = jnp.bitwise_and(pv, 0xFFFF)
            dv = jnp.right_shift(pv, 16)
            w = (ew_v[ci, pl.ds(g * 16, 16)] - mn) * inv
            dis = plsc.load_gather(dinv_v, [sv])
            did = plsc.load_gather(dinv_v, [dv])
            ew_v[ci, pl.ds(g * 16, 16)] = dis * w * did

    unpack(0, sb0_v, db0_v)
    unpack(1, sb1_v, db1_v)
    pltpu.async_copy(x_hbm.at[sb0_v], rows0_v, gsem0)
    pltpu.async_copy(x_hbm.at[sb1_v], rows1_v, gsem1)

    plsc.subcore_barrier()

    NPAIR = (NCHUNK - 1) // 2

    @pl.loop(0, NPAIR)
    def _pair(k):
        ci0 = 2 * k
        ci1 = 2 * k + 1
        pltpu.make_async_copy(x_hbm.at[sb0_v], rows0_v, gsem0).wait()
        scale(rows0_v, ci0)
        s0 = pltpu.async_copy(rows0_v, y_sh.at[db0_v], ssem0, add=True)
        pltpu.make_async_copy(x_hbm.at[sb1_v], rows1_v, gsem1).wait()
        scale(rows1_v, ci1)
        s1 = pltpu.async_copy(rows1_v, y_sh.at[db1_v], ssem1, add=True)
        s0.wait()
        unpack(ci0 + 2, sb0_v, db0_v)
        pltpu.async_copy(x_hbm.at[sb0_v], rows0_v, gsem0)
        s1.wait()

        @pl.when(k < NPAIR - 1)
        def _refill_odd():
            unpack(ci1 + 2, sb1_v, db1_v)
            pltpu.async_copy(x_hbm.at[sb1_v], rows1_v, gsem1)

    pltpu.make_async_copy(x_hbm.at[sb0_v], rows0_v, gsem0).wait()
    scale(rows0_v, NCHUNK - 1)
    pltpu.async_copy(rows0_v, y_sh.at[db0_v], ssem0, add=True).wait()

    plsc.subcore_barrier()

    pltpu.sync_copy(y_sh.at[pl.ds(s * ROWS_PT, ROWS_PT)],
                    ypart_hbm.at[c, pl.ds(s * ROWS_PT, ROWS_PT)])


def _k4(x, pk, ew, mn16, inv16, dinv):
    mesh = plsc.VectorSubcoreMesh(core_axis_name="c", subcore_axis_name="s",
                                  num_cores=NC, num_subcores=NS)
    kfn = pl.kernel(
        _k4_body,
        out_type=jax.ShapeDtypeStruct((NC, N, D), f32),
        mesh=mesh,
        compiler_params=pltpu.CompilerParams(needs_layout_passes=False, use_tc_tiling_on_sc=False),
        scratch_types=[
            pltpu.VMEM((N,), f32),
            pltpu.VMEM((NCHUNK, CHUNK), i32),
            pltpu.VMEM((NCHUNK, CHUNK), f32),
            pltpu.VMEM((CHUNK, D), f32),
            pltpu.VMEM((CHUNK, D), f32),
            pltpu.VMEM((CHUNK,), i32),
            pltpu.VMEM((CHUNK,), i32),
            pltpu.VMEM((CHUNK,), i32),
            pltpu.VMEM((CHUNK,), i32),
            pltpu.VMEM((16,), f32),
            pltpu.VMEM((16,), f32),
            pltpu.VMEM_SHARED((N, D), f32),
            pltpu.SemaphoreType.DMA,
            pltpu.SemaphoreType.DMA,
            pltpu.SemaphoreType.DMA,
            pltpu.SemaphoreType.DMA,
        ],
    )
    return kfn(x.reshape(N, D),
               pk.reshape(NW, NCHUNK, CHUNK),
               ew.reshape(NW, NCHUNK, CHUNK),
               mn16, inv16, dinv)


def _k5_body(yp_ref, x_ref, dinv_ref, w0_ref, b0_ref, w1_ref, b1_ref,
             w2_ref, b2_ref, out_ref):
    d = dinv_ref[...]
    y = yp_ref[0] + yp_ref[1] + x_ref[...] * (d * d)
    out_ref[:, 0:D] = jnp.dot(y, w0_ref[...], preferred_element_type=f32) + b0_ref[...]
    out_ref[:, D:2 * D] = jnp.dot(y, w1_ref[...], preferred_element_type=f32) + b1_ref[...]
    out_ref[:, 2 * D:3 * D] = jnp.dot(y, w2_ref[...], preferred_element_type=f32) + b2_ref[...]


def _k5(ypart, x, dinv2d, W0, b0, W1, b1, W2, b2):
    R = 1000
    grid = N // R
    wspec = pl.BlockSpec((D, D), lambda i: (0, 0))
    bspec = pl.BlockSpec((1, D), lambda i: (0, 0))
    return pl.pallas_call(
        _k5_body,
        grid=(grid,),
        in_specs=[
            pl.BlockSpec((NC, R, D), lambda i: (0, i, 0)),
            pl.BlockSpec((R, D), lambda i: (i, 0)),
            pl.BlockSpec((R, 1), lambda i: (i, 0)),
            wspec, bspec, wspec, bspec, wspec, bspec,
        ],
        out_specs=pl.BlockSpec((R, 3 * D), lambda i: (i, 0)),
        out_shape=jax.ShapeDtypeStruct((N, 3 * D), f32),
    )(ypart, x, dinv2d, W0, b0, W1, b1, W2, b2)


def kernel(x, edge_index, edge_weight, W0, b0, W1, b1, W2, b2):
    src = edge_index[0]
    dst = edge_index[1]

    mn, inv = _k1(edge_weight.reshape(E // D, D))
    mn16 = jnp.broadcast_to(mn.reshape(()), (16,))
    inv16 = jnp.broadcast_to(inv.reshape(()), (16,))

    degp = _k2(dst, edge_weight, mn16, inv16)
    dinv2d = _k3(degp)
    dinv = dinv2d.reshape(N)

    pk = jnp.bitwise_or(jnp.left_shift(dst, 16), src)
    ypart = _k4(x, pk, edge_weight, mn16, inv16, dinv)

    return _k5(ypart, x, dinv2d.reshape(N, 1), W0,
               b0.reshape(1, D), W1, b1.reshape(1, D), W2, b2.reshape(1, D))
